# baseline (speedup 1.0000x reference)
import functools
import math

import jax
import jax.numpy as jnp
from jax.experimental import pallas as pl
from jax.experimental.pallas import tpu as pltpu

_LB_COEFF = 0.01
_INV_E = 1.0 / math.e


def _moe_body(K, T, E, O,
              x_ref, gate_w_ref, gate_b_ref, W1_ref, b1_ref,
              W2_ref, b2_ref,
              alpha_b_ref, camp_ref, tension_ref, ln_g_ref,
              ln_b_ref, out_ref, lb_ref,
              ws_s, mr_s, fp_s):
    RO = 16
    t = pl.program_id(0)
    e = pl.program_id(1)
    hh = pl.program_id(2)
    nt = pl.num_programs(0)
    B_total = nt * T

    @pl.when((e == 0) & (hh == 0))
    def _gate():
        rawT = jax.lax.dot_general(
            gate_w_ref[...], x_ref[...], (((1,), (1,)), ((), ())),
            preferred_element_type=jnp.float32)
        scoresT = (rawT[:E, :] + gate_b_ref[...]) * _INV_E
        m = jnp.max(scoresT, axis=0, keepdims=True)
        ex = jnp.exp(scoresT - m)
        probsT = ex / jnp.sum(ex, axis=0, keepdims=True)

        row = jax.lax.broadcasted_iota(jnp.int32, (E, T), 0)
        rank = jnp.zeros((E, T), jnp.float32)
        for ep in range(E):
            pe = probsT[ep:ep + 1, :]
            beats = (pe > probsT) | ((pe == probsT) & (ep < row))
            rank = rank + beats.astype(jnp.float32)
        maskf = (rank < K).astype(jnp.float32)

        w = probsT * maskf
        wn = w / (jnp.sum(w, axis=0, keepdims=True) + 1e-8)
        s_col = 2.0 * jax.nn.sigmoid(camp_ref[...]) - 1.0
        mix8 = jnp.broadcast_to(rawT[E:E + 1, :], (E, T))
        stacked = jnp.concatenate([wn, wn * s_col, mix8], axis=0)
        for c in range(T // 256):
            slg = slice(c * 256, (c + 1) * 256)
            ws_s[slg, :] = stacked[:, slg].T
        mr_s[...] = jnp.zeros_like(mr_s)

        fsum = jnp.sum((wn > 0).astype(jnp.float32), axis=1, keepdims=True)
        psum = jnp.sum(probsT, axis=1, keepdims=True)

        @pl.when(t == 0)
        def _init_fp():
            fp_s[:, 0:1] = fsum
            fp_s[:, 1:2] = psum

        @pl.when(t != 0)
        def _acc_fp():
            fp_s[:, 0:1] = fp_s[:, 0:1] + fsum
            fp_s[:, 1:2] = fp_s[:, 1:2] + psum

    lane = jax.lax.broadcasted_iota(jnp.int32, (1, E), 1)
    C = T // 256
    Tc = T // C
    w1 = W1_ref[0, 0]
    oh = (lane == e).astype(jnp.float32)
    w_e = jnp.sum(ws_s[:, 0:E] * oh, axis=-1, keepdims=True)
    sw_e = jnp.sum(ws_s[:, E:2 * E] * oh, axis=-1, keepdims=True)
    b2row = (hh == 0).astype(jnp.float32) * b2_ref[0]
    for c in range(C):
        sl = slice(c * Tc, (c + 1) * Tc)
        h = jax.lax.dot_general(x_ref[sl, :], w1,
                                (((1,), (1,)), ((), ())),
                                preferred_element_type=jnp.float32)
        h = jnp.maximum(h + b1_ref[0, 0], 0.0)
        e_out = jax.lax.dot_general(h, W2_ref[0, 0],
                                    (((1,), (1,)), ((), ())),
                                    preferred_element_type=jnp.float32)
        e_out = e_out + b2row
        mr_s[sl, 0:O] = mr_s[sl, 0:O] + w_e[sl, :] * e_out
        mr_s[sl, RO:RO + O] = mr_s[sl, RO:RO + O] + sw_e[sl, :] * e_out

    @pl.when((e == E - 1) & (hh == 1))
    def _epilogue():
        for c in range(C):
            sl = slice(c * Tc, (c + 1) * Tc)
            moe = mr_s[sl, 0:O]
            rep = mr_s[sl, RO:RO + O]
            sq = rep * rep
            tension = jnp.mean(sq, axis=-1, keepdims=True)
            norm = jnp.sqrt(jnp.sum(sq, axis=-1, keepdims=True))
            direction = rep / (norm + 1e-8)
            t_out = tension_ref[0, 0] * jnp.sqrt(tension + 1e-8) * direction
            mu = jnp.mean(t_out, axis=-1, keepdims=True)
            var = jnp.mean((t_out - mu) ** 2, axis=-1, keepdims=True)
            t_out = ((t_out - mu) / jnp.sqrt(var + 1e-5)) * ln_g_ref[...] \
                + ln_b_ref[...]
            mix = jax.nn.sigmoid(ws_s[sl, 2 * E:2 * E + 1]
                                 + alpha_b_ref[0, 0])
            out_ref[sl, :] = mix * moe + (1.0 - mix) * t_out

        @pl.when(t == nt - 1)
        def _lb():
            f = fp_s[:, 0:1] / B_total
            P = fp_s[:, 1:2] / B_total
            lb_ref[0, 0] = _LB_COEFF * E * jnp.sum(f * P)


def kernel(x, gate_w, gate_b, W1, b1, W2, b2, alpha_w, alpha_b,
           camp_logits, tension_scale, ln_gamma, ln_beta):
    B, D = x.shape
    E, H, _ = W1.shape
    O = W2.shape[1]
    K = max(1, int(E * 0.625))
    T = 2048
    nt = B // T

    H2 = H // 2
    W1h = W1.reshape(E, 2, H2, D)
    W2h = jnp.transpose(W2.reshape(E, O, 2, H2), (0, 2, 1, 3))
    b1r = b1.reshape(E, 2, 1, H2)
    b2r = b2.reshape(E, 1, O)
    gate_w_aug = jnp.concatenate([gate_w, alpha_w], axis=0)
    gate_b2 = gate_b.reshape(E, 1)
    alpha_b2 = alpha_b.reshape(1, 1)
    camp2 = camp_logits.reshape(E, 1)
    tension2 = tension_scale.reshape(1, 1)
    ln_g2 = ln_gamma.reshape(1, O)
    ln_b2 = ln_beta.reshape(1, O)

    body = functools.partial(_moe_body, K, T, E, O)
    full = lambda shape: pl.BlockSpec(shape, lambda t, e, hh: (0,) * len(shape))

    out, lb = pl.pallas_call(
        body,
        grid=(nt, E, 2),
        in_specs=[
            pl.BlockSpec((T, D), lambda t, e, hh: (t, 0)),
            full((E + 1, D)),
            full((E, 1)),
            pl.BlockSpec((1, 1, H2, D),
                         lambda t, e, hh: (e, hh, 0, 0)),
            pl.BlockSpec((1, 1, 1, H2),
                         lambda t, e, hh: (e, hh, 0, 0)),
            pl.BlockSpec((1, 1, O, H2),
                         lambda t, e, hh: (e, hh, 0, 0)),
            pl.BlockSpec((1, 1, O), lambda t, e, hh: (e, 0, 0)),
            pl.BlockSpec(memory_space=pltpu.SMEM),
            full((E, 1)),
            pl.BlockSpec(memory_space=pltpu.SMEM),
            full((1, O)),
            full((1, O)),
        ],
        out_specs=[
            pl.BlockSpec((T, O), lambda t, e, hh: (t, 0)),
            pl.BlockSpec(memory_space=pltpu.SMEM),
        ],
        out_shape=[
            jax.ShapeDtypeStruct((B, O), jnp.float32),
            jax.ShapeDtypeStruct((1, 1), jnp.float32),
        ],
        scratch_shapes=[
            pltpu.VMEM((T, 3 * E), jnp.float32),
            pltpu.VMEM((T, 32), jnp.float32),
            pltpu.VMEM((E, 2), jnp.float32),
        ],
        compiler_params=pltpu.CompilerParams(
            dimension_semantics=("arbitrary", "arbitrary", "arbitrary")),
    )(x, gate_w_aug, gate_b2, W1h, b1r, W2h, b2r, alpha_b2, camp2,
      tension2, ln_g2, ln_b2)
    return out, lb[0, 0]


# speedup vs baseline: 1.1084x; 1.1084x over previous
import functools
import math

import jax
import jax.numpy as jnp
from jax.experimental import pallas as pl
from jax.experimental.pallas import tpu as pltpu

_LB_COEFF = 0.01
_INV_E = 1.0 / math.e


def _moe_body(K, T, E, O,
              x_ref, gate_w_ref, gate_b_ref, W1_ref, b1_ref,
              W2_ref, b2_ref,
              alpha_b_ref, camp_ref, tension_ref, ln_g_ref,
              ln_b_ref, out_ref, lb_ref,
              ws_s, moe_s, rep_s, fp_s):
    t = pl.program_id(0)
    e = pl.program_id(1)
    nt = pl.num_programs(0)
    B_total = nt * T

    @pl.when(e == 0)
    def _gate():
        rawT = jax.lax.dot_general(
            gate_w_ref[...], x_ref[...], (((1,), (1,)), ((), ())),
            preferred_element_type=jnp.float32)
        scoresT = (rawT[:E, :] + gate_b_ref[...]) * _INV_E
        m = jnp.max(scoresT, axis=0, keepdims=True)
        ex = jnp.exp(scoresT - m)
        probsT = ex / jnp.sum(ex, axis=0, keepdims=True)

        row = jax.lax.broadcasted_iota(jnp.int32, (E, T), 0)
        rank = jnp.zeros((E, T), jnp.float32)
        for ep in range(E):
            pe = probsT[ep:ep + 1, :]
            beats = (pe > probsT) | ((pe == probsT) & (ep < row))
            rank = rank + beats.astype(jnp.float32)
        maskf = (rank < K).astype(jnp.float32)

        w = probsT * maskf
        wn = w / (jnp.sum(w, axis=0, keepdims=True) + 1e-8)
        s_col = 2.0 * jax.nn.sigmoid(camp_ref[...]) - 1.0
        mix8 = jnp.broadcast_to(rawT[E:E + 1, :], (E, T))
        stacked = jnp.concatenate([wn, wn * s_col, mix8], axis=0)
        ws_s[...] = stacked.T
        moe_s[...] = jnp.zeros_like(moe_s)
        rep_s[...] = jnp.zeros_like(rep_s)

        fsum = jnp.sum((wn > 0).astype(jnp.float32), axis=1, keepdims=True)
        psum = jnp.sum(probsT, axis=1, keepdims=True)

        @pl.when(t == 0)
        def _init_fp():
            fp_s[:, 0:1] = fsum
            fp_s[:, 1:2] = psum

        @pl.when(t != 0)
        def _acc_fp():
            fp_s[:, 0:1] = fp_s[:, 0:1] + fsum
            fp_s[:, 1:2] = fp_s[:, 1:2] + psum

    w1 = W1_ref[0]
    lane = jax.lax.broadcasted_iota(jnp.int32, (1, E), 1)
    oh = (lane == e).astype(jnp.float32)
    w_e = jnp.sum(ws_s[:, 0:E] * oh, axis=-1, keepdims=True)
    sw_e = jnp.sum(ws_s[:, E:2 * E] * oh, axis=-1, keepdims=True)

    C = T // 256
    Tc = T // C
    for c in range(C):
        sl = slice(c * Tc, (c + 1) * Tc)
        h = jax.lax.dot_general(x_ref[sl, :], w1, (((1,), (1,)), ((), ())),
                                preferred_element_type=jnp.float32)
        h = jnp.maximum(h + b1_ref[0], 0.0)
        e_out = jax.lax.dot_general(h, W2_ref[0], (((1,), (1,)), ((), ())),
                                    preferred_element_type=jnp.float32)
        e_out = e_out + b2_ref[0]
        moe_s[sl, :] = moe_s[sl, :] + w_e[sl, :] * e_out
        rep_s[sl, :] = rep_s[sl, :] + sw_e[sl, :] * e_out

    @pl.when(e == E - 1)
    def _epilogue():
        moe = moe_s[...]
        rep = rep_s[...]
        sq = rep * rep
        tension = jnp.mean(sq, axis=-1, keepdims=True)
        norm = jnp.sqrt(jnp.sum(sq, axis=-1, keepdims=True))
        direction = rep / (norm + 1e-8)
        t_out = tension_ref[0, 0] * jnp.sqrt(tension + 1e-8) * direction
        mu = jnp.mean(t_out, axis=-1, keepdims=True)
        var = jnp.mean((t_out - mu) ** 2, axis=-1, keepdims=True)
        t_out = ((t_out - mu) / jnp.sqrt(var + 1e-5)) * ln_g_ref[...] \
            + ln_b_ref[...]
        mix = jax.nn.sigmoid(ws_s[:, 2 * E:2 * E + 1] + alpha_b_ref[0, 0])
        out_ref[...] = mix * moe + (1.0 - mix) * t_out

        @pl.when(t == nt - 1)
        def _lb():
            f = fp_s[:, 0:1] / B_total
            P = fp_s[:, 1:2] / B_total
            lb_ref[0, 0] = _LB_COEFF * E * jnp.sum(f * P)


def kernel(x, gate_w, gate_b, W1, b1, W2, b2, alpha_w, alpha_b,
           camp_logits, tension_scale, ln_gamma, ln_beta):
    B, D = x.shape
    E, H, _ = W1.shape
    O = W2.shape[1]
    K = max(1, int(E * 0.625))
    T = 1024
    nt = B // T

    b1r = b1.reshape(E, 1, H)
    b2r = b2.reshape(E, 1, O)
    gate_w_aug = jnp.concatenate([gate_w, alpha_w], axis=0)
    gate_b2 = gate_b.reshape(E, 1)
    alpha_b2 = alpha_b.reshape(1, 1)
    camp2 = camp_logits.reshape(E, 1)
    tension2 = tension_scale.reshape(1, 1)
    ln_g2 = ln_gamma.reshape(1, O)
    ln_b2 = ln_beta.reshape(1, O)

    body = functools.partial(_moe_body, K, T, E, O)
    full = lambda shape: pl.BlockSpec(shape, lambda t, e: (0,) * len(shape))

    out, lb = pl.pallas_call(
        body,
        grid=(nt, E),
        in_specs=[
            pl.BlockSpec((T, D), lambda t, e: (t, 0)),
            full((E + 1, D)),
            full((E, 1)),
            pl.BlockSpec((1, H, D), lambda t, e: (e, 0, 0)),
            pl.BlockSpec((1, 1, H), lambda t, e: (e, 0, 0)),
            pl.BlockSpec((1, O, H), lambda t, e: (e, 0, 0)),
            pl.BlockSpec((1, 1, O), lambda t, e: (e, 0, 0)),
            pl.BlockSpec(memory_space=pltpu.SMEM),
            full((E, 1)),
            pl.BlockSpec(memory_space=pltpu.SMEM),
            full((1, O)),
            full((1, O)),
        ],
        out_specs=[
            pl.BlockSpec((T, O), lambda t, e: (t, 0)),
            pl.BlockSpec(memory_space=pltpu.SMEM),
        ],
        out_shape=[
            jax.ShapeDtypeStruct((B, O), jnp.float32),
            jax.ShapeDtypeStruct((1, 1), jnp.float32),
        ],
        scratch_shapes=[
            pltpu.VMEM((T, 3 * E), jnp.float32),
            pltpu.VMEM((T, O), jnp.float32),
            pltpu.VMEM((T, O), jnp.float32),
            pltpu.VMEM((E, 2), jnp.float32),
        ],
        compiler_params=pltpu.CompilerParams(
            dimension_semantics=("arbitrary", "arbitrary")),
    )(x, gate_w_aug, gate_b2, W1, b1r, W2, b2r, alpha_b2, camp2,
      tension2, ln_g2, ln_b2)
    return out, lb[0, 0]
